# Initial kernel scaffold; baseline (speedup 1.0000x reference)
import jax
import jax.numpy as jnp
from jax.experimental import pallas as pl


def kernel(encoded_outs, encoded_lens, emb, Wx, Wh, b, W_enc, W_pred, b_joint, W_out, b_out):
    raise NotImplementedError("write your pallas kernel here")


# speedup vs baseline: 4.4648x; 4.4648x over previous
import functools

import jax
import jax.numpy as jnp
from jax.experimental import pallas as pl
from jax.experimental.pallas import tpu as pltpu

_U = 4
_CHUNK = 2048


def _round_up(x, m):
    return (x + m - 1) // m * m


def _matmul_block_kernel(a_ref, b_ref, o_ref):
    o_ref[...] = jnp.dot(a_ref[...], b_ref[...], preferred_element_type=jnp.float32)


def _block_matmul(a, b, block_m):
    m, k = a.shape
    n = b.shape[1]
    return pl.pallas_call(
        _matmul_block_kernel,
        grid=(m // block_m,),
        in_specs=[
            pl.BlockSpec((block_m, k), lambda i: (i, 0)),
            pl.BlockSpec((k, n), lambda i: (0, 0)),
        ],
        out_specs=pl.BlockSpec((block_m, n), lambda i: (i, 0)),
        out_shape=jax.ShapeDtypeStruct((m, n), jnp.float32),
        compiler_params=pltpu.CompilerParams(
            dimension_semantics=("parallel",),
            vmem_limit_bytes=60 * 1024 * 1024,
        ),
    )(a, b)


def _decode_kernel(
    lens_ref,
    encp_ref,
    b_ref,
    bj_ref,
    bo_ref,
    wh_ref,
    wp_ref,
    wo_ref,
    embwx_hbm,
    out_ref,
    h_out_ref,
    c_out_ref,
    xwx_ref,
    h_ref,
    c_ref,
    last_ref,
    sem,
    *,
    blank, b_sz, h_sz, np_sz,
):
    out_ref[...] = jnp.full(out_ref.shape, blank, jnp.int32)
    h_ref[...] = jnp.zeros_like(h_ref)
    c_ref[...] = jnp.zeros_like(c_ref)
    last_ref[...] = jnp.full(last_ref.shape, blank, jnp.int32)
    xwx_ref[...] = jnp.zeros_like(xwx_ref)

    max_len = jnp.max(lens_ref[...].astype(jnp.float32)).astype(jnp.int32)
    n_steps = _U * max_len

    sub8 = jax.lax.broadcasted_iota(jnp.int32, (b_sz, b_sz), 0)
    lane8 = jax.lax.broadcasted_iota(jnp.int32, (b_sz, b_sz), 1)

    def to_lanes(v):
        m = jnp.where(sub8 == lane8, jnp.broadcast_to(v, (b_sz, b_sz)), 0)
        return jnp.sum(m, axis=0, keepdims=True)

    def body(s, carry):
        t = s // _U

        @pl.when(s > 0)
        def _wait_gather():
            for i in range(b_sz):
                pltpu.make_async_copy(xwx_ref.at[i], xwx_ref.at[i], sem).wait()

        h = h_ref[...]
        c = c_ref[...]
        last = last_ref[...]

        hb = h.astype(jnp.bfloat16)
        z = (xwx_ref[...] + jnp.dot(hb, wh_ref[...], preferred_element_type=jnp.float32)) + b_ref[...]
        i_g = jax.nn.sigmoid(z[:, :h_sz])
        f_g = jax.nn.sigmoid(z[:, h_sz:2 * h_sz])
        g_g = jnp.tanh(z[:, 2 * h_sz:3 * h_sz])
        o_g = jax.nn.sigmoid(z[:, 3 * h_sz:])
        c2 = f_g * c + i_g * g_g
        h2 = o_g * jnp.tanh(c2)

        enc_t = encp_ref[pl.ds(t, 1), :, :].reshape(b_sz, -1)
        r = jnp.dot(h2.astype(jnp.bfloat16), wp_ref[...], preferred_element_type=jnp.float32)
        q = jnp.tanh((enc_t + r) + bj_ref[...])
        qb = q.astype(jnp.bfloat16)

        best_m = None
        best_i = None
        for c0 in range(0, np_sz, _CHUNK):
            c1 = min(c0 + _CHUNK, np_sz)
            part = (
                jnp.dot(qb, wo_ref[:, c0:c1], preferred_element_type=jnp.float32)
                + bo_ref[:, c0:c1]
            )
            m = jnp.max(part, axis=-1, keepdims=True)
            iota_f = jax.lax.broadcasted_iota(jnp.int32, (b_sz, c1 - c0), 1).astype(jnp.float32)
            mi = jnp.min(
                jnp.where(part == m, iota_f, jnp.float32(np_sz)), axis=-1, keepdims=True
            ) + jnp.float32(c0)
            if best_m is None:
                best_m, best_i = m, mi
            else:
                better = m > best_m
                best_i = jnp.where(better, mi, best_i)
                best_m = jnp.maximum(best_m, m)
        sym = best_i.astype(jnp.int32)

        active = t < lens_ref[...]
        out_lab = jnp.where(active, sym, blank)
        advance = jnp.logical_and(active, sym != blank)
        new_last = jnp.where(advance, sym, last)
        h_ref[...] = jnp.where(advance, h2, h)
        c_ref[...] = jnp.where(advance, c2, c)
        last_ref[...] = new_last

        out_ref[pl.ds(s, 1)] = to_lanes(out_lab).reshape(1, 1, b_sz)

        nl = to_lanes(new_last)
        for i in range(b_sz):
            pltpu.make_async_copy(embwx_hbm.at[nl[0, i]], xwx_ref.at[i], sem).start()
        return 0

    jax.lax.fori_loop(0, n_steps, body, 0)

    for i in range(b_sz):
        pltpu.make_async_copy(xwx_ref.at[i], xwx_ref.at[i], sem).wait()

    h_out_ref[...] = h_ref[...]
    c_out_ref[...] = c_ref[...]


def kernel(encoded_outs, encoded_lens, emb, Wx, Wh, b, W_enc, W_pred, b_joint, W_out, b_out):
    B, D, T = encoded_outs.shape
    H = Wh.shape[0]
    J = W_enc.shape[1]
    V1 = emb.shape[0]
    blank = V1 - 1

    vpr = _round_up(V1, 128)
    emb_b = jnp.pad(emb, ((0, vpr - V1), (0, 0))).astype(jnp.bfloat16)
    embwx = _block_matmul(emb_b, Wx.astype(jnp.bfloat16), 128)

    enc_tm = jnp.transpose(encoded_outs, (2, 0, 1)).reshape(T * B, D)
    tbp = _round_up(T * B, 128)
    enc_tm = jnp.pad(enc_tm, ((0, tbp - T * B), (0, 0))).astype(jnp.bfloat16)
    encp = _block_matmul(enc_tm, W_enc.astype(jnp.bfloat16), 128)[: T * B]
    encp = encp.reshape(T, B, J)

    np_sz = _round_up(V1, 256)
    wo_b = jnp.pad(W_out, ((0, 0), (0, np_sz - V1))).astype(jnp.bfloat16)
    bo_p = jnp.pad(b_out, (0, np_sz - V1), constant_values=float("-inf")).reshape(1, np_sz)

    decode = pl.pallas_call(
        functools.partial(_decode_kernel, blank=blank, b_sz=B, h_sz=H, np_sz=np_sz),
        in_specs=[
            pl.BlockSpec(memory_space=pltpu.VMEM),
            pl.BlockSpec(memory_space=pltpu.VMEM),
            pl.BlockSpec(memory_space=pltpu.VMEM),
            pl.BlockSpec(memory_space=pltpu.VMEM),
            pl.BlockSpec(memory_space=pltpu.VMEM),
            pl.BlockSpec(memory_space=pltpu.VMEM),
            pl.BlockSpec(memory_space=pltpu.VMEM),
            pl.BlockSpec(memory_space=pltpu.VMEM),
            pl.BlockSpec(memory_space=pl.ANY),
        ],
        out_shape=(
            jax.ShapeDtypeStruct((T * _U, 1, B), jnp.int32),
            jax.ShapeDtypeStruct((B, H), jnp.float32),
            jax.ShapeDtypeStruct((B, H), jnp.float32),
        ),
        scratch_shapes=[
            pltpu.VMEM((B, 4 * H), jnp.float32),
            pltpu.VMEM((B, H), jnp.float32),
            pltpu.VMEM((B, H), jnp.float32),
            pltpu.VMEM((B, 1), jnp.int32),
            pltpu.SemaphoreType.DMA,
        ],
        compiler_params=pltpu.CompilerParams(
            vmem_limit_bytes=60 * 1024 * 1024,
        ),
    )
    labels_flat, h_fin, c_fin = decode(
        encoded_lens.reshape(B, 1).astype(jnp.int32),
        encp,
        b.reshape(1, 4 * H),
        b_joint.reshape(1, J),
        bo_p,
        Wh.astype(jnp.bfloat16),
        W_pred.astype(jnp.bfloat16),
        wo_b,
        embwx,
    )
    labels = labels_flat.reshape(T, _U, B).transpose(2, 0, 1)
    return labels, h_fin, c_fin


# speedup vs baseline: 5.0275x; 1.1260x over previous
import functools

import jax
import jax.numpy as jnp
from jax.experimental import pallas as pl
from jax.experimental.pallas import tpu as pltpu

_U = 4
_CHUNK = 2048


def _round_up(x, m):
    return (x + m - 1) // m * m


def _matmul_block_kernel(a_ref, b_ref, o_ref):
    o_ref[...] = jnp.dot(a_ref[...], b_ref[...], preferred_element_type=jnp.float32)


def _block_matmul(a, b, block_m):
    m, k = a.shape
    n = b.shape[1]
    return pl.pallas_call(
        _matmul_block_kernel,
        grid=(m // block_m,),
        in_specs=[
            pl.BlockSpec((block_m, k), lambda i: (i, 0)),
            pl.BlockSpec((k, n), lambda i: (0, 0)),
        ],
        out_specs=pl.BlockSpec((block_m, n), lambda i: (i, 0)),
        out_shape=jax.ShapeDtypeStruct((m, n), jnp.float32),
        compiler_params=pltpu.CompilerParams(
            dimension_semantics=("parallel",),
            vmem_limit_bytes=60 * 1024 * 1024,
        ),
    )(a, b)


def _decode_kernel(
    lens_ref,
    encp_ref,
    b_ref,
    bj_ref,
    bo_ref,
    wh_ref,
    wp_ref,
    wo_ref,
    embwx_hbm,
    out_ref,
    h_out_ref,
    c_out_ref,
    xwx_ref,
    h_ref,
    c_ref,
    last_ref,
    sem,
    *,
    blank, b_sz, h_sz, np_sz,
):
    out_ref[...] = jnp.full(out_ref.shape, blank, jnp.int32)
    h_ref[...] = jnp.zeros_like(h_ref)
    c_ref[...] = jnp.zeros_like(c_ref)
    last_ref[...] = jnp.full(last_ref.shape, blank, jnp.int32)
    xwx_ref[...] = jnp.zeros_like(xwx_ref)

    max_len = jnp.max(lens_ref[...].astype(jnp.float32)).astype(jnp.int32)

    sub8 = jax.lax.broadcasted_iota(jnp.int32, (b_sz, b_sz), 0)
    lane8 = jax.lax.broadcasted_iota(jnp.int32, (b_sz, b_sz), 1)

    def to_lanes(v):
        m = jnp.where(sub8 == lane8, jnp.broadcast_to(v, (b_sz, b_sz)), 0)
        return jnp.sum(m, axis=0, keepdims=True)

    def step(t, u, enc_t, active, wait_pred):
        h = h_ref[...]
        c = c_ref[...]
        last = last_ref[...]

        hb = h.astype(jnp.bfloat16)
        wh_part = jnp.dot(hb, wh_ref[...], preferred_element_type=jnp.float32)
        if wait_pred is None:
            for i in range(b_sz):
                pltpu.make_async_copy(xwx_ref.at[i], xwx_ref.at[i], sem).wait()
        elif wait_pred is not False:
            @pl.when(wait_pred)
            def _wait_gather():
                for i in range(b_sz):
                    pltpu.make_async_copy(xwx_ref.at[i], xwx_ref.at[i], sem).wait()
        z = (xwx_ref[...] + wh_part) + b_ref[...]
        i_g = jax.nn.sigmoid(z[:, :h_sz])
        f_g = jax.nn.sigmoid(z[:, h_sz:2 * h_sz])
        g_g = jnp.tanh(z[:, 2 * h_sz:3 * h_sz])
        o_g = jax.nn.sigmoid(z[:, 3 * h_sz:])
        c2 = f_g * c + i_g * g_g
        h2 = o_g * jnp.tanh(c2)

        r = jnp.dot(h2.astype(jnp.bfloat16), wp_ref[...], preferred_element_type=jnp.float32)
        q = jnp.tanh((enc_t + r) + bj_ref[...])
        qb = q.astype(jnp.bfloat16)

        best_m = None
        best_i = None
        for c0 in range(0, np_sz, _CHUNK):
            c1 = min(c0 + _CHUNK, np_sz)
            part = (
                jnp.dot(qb, wo_ref[:, c0:c1], preferred_element_type=jnp.float32)
                + bo_ref[:, c0:c1]
            )
            m = jnp.max(part, axis=-1, keepdims=True)
            iota_f = jax.lax.broadcasted_iota(jnp.int32, (b_sz, c1 - c0), 1).astype(jnp.float32)
            mi = jnp.min(
                jnp.where(part == m, iota_f, jnp.float32(np_sz)), axis=-1, keepdims=True
            ) + jnp.float32(c0)
            if best_m is None:
                best_m, best_i = m, mi
            else:
                better = m > best_m
                best_i = jnp.where(better, mi, best_i)
                best_m = jnp.maximum(best_m, m)
        sym = best_i.astype(jnp.int32)

        out_lab = jnp.where(active, sym, blank)
        advance = jnp.logical_and(active, sym != blank)
        new_last = jnp.where(advance, sym, last)
        h_ref[...] = jnp.where(advance, h2, h)
        c_ref[...] = jnp.where(advance, c2, c)
        last_ref[...] = new_last

        out_ref[pl.ds(_U * t + u, 1)] = to_lanes(out_lab).reshape(1, 1, b_sz)

        nl = to_lanes(new_last)
        for i in range(b_sz):
            pltpu.make_async_copy(embwx_hbm.at[nl[0, i]], xwx_ref.at[i], sem).start()

    def body(t, carry):
        enc_t = encp_ref[pl.ds(t, 1), :, :].reshape(b_sz, -1)
        active = t < lens_ref[...]
        step(t, 0, enc_t, active, wait_pred=t > 0)
        for u in range(1, _U):
            step(t, u, enc_t, active, wait_pred=None)
        return 0

    jax.lax.fori_loop(0, max_len, body, 0)

    for i in range(b_sz):
        pltpu.make_async_copy(xwx_ref.at[i], xwx_ref.at[i], sem).wait()

    h_out_ref[...] = h_ref[...]
    c_out_ref[...] = c_ref[...]


def kernel(encoded_outs, encoded_lens, emb, Wx, Wh, b, W_enc, W_pred, b_joint, W_out, b_out):
    B, D, T = encoded_outs.shape
    H = Wh.shape[0]
    J = W_enc.shape[1]
    V1 = emb.shape[0]
    blank = V1 - 1

    vpr = _round_up(V1, 128)
    emb_b = jnp.pad(emb, ((0, vpr - V1), (0, 0))).astype(jnp.bfloat16)
    embwx = _block_matmul(emb_b, Wx.astype(jnp.bfloat16), 128)

    enc_tm = jnp.transpose(encoded_outs, (2, 0, 1)).reshape(T * B, D)
    tbp = _round_up(T * B, 128)
    enc_tm = jnp.pad(enc_tm, ((0, tbp - T * B), (0, 0))).astype(jnp.bfloat16)
    encp = _block_matmul(enc_tm, W_enc.astype(jnp.bfloat16), 128)[: T * B]
    encp = encp.reshape(T, B, J)

    np_sz = _round_up(V1, 256)
    wo_b = jnp.pad(W_out, ((0, 0), (0, np_sz - V1))).astype(jnp.bfloat16)
    bo_p = jnp.pad(b_out, (0, np_sz - V1), constant_values=float("-inf")).reshape(1, np_sz)

    decode = pl.pallas_call(
        functools.partial(_decode_kernel, blank=blank, b_sz=B, h_sz=H, np_sz=np_sz),
        in_specs=[
            pl.BlockSpec(memory_space=pltpu.VMEM),
            pl.BlockSpec(memory_space=pltpu.VMEM),
            pl.BlockSpec(memory_space=pltpu.VMEM),
            pl.BlockSpec(memory_space=pltpu.VMEM),
            pl.BlockSpec(memory_space=pltpu.VMEM),
            pl.BlockSpec(memory_space=pltpu.VMEM),
            pl.BlockSpec(memory_space=pltpu.VMEM),
            pl.BlockSpec(memory_space=pltpu.VMEM),
            pl.BlockSpec(memory_space=pl.ANY),
        ],
        out_shape=(
            jax.ShapeDtypeStruct((T * _U, 1, B), jnp.int32),
            jax.ShapeDtypeStruct((B, H), jnp.float32),
            jax.ShapeDtypeStruct((B, H), jnp.float32),
        ),
        scratch_shapes=[
            pltpu.VMEM((B, 4 * H), jnp.float32),
            pltpu.VMEM((B, H), jnp.float32),
            pltpu.VMEM((B, H), jnp.float32),
            pltpu.VMEM((B, 1), jnp.int32),
            pltpu.SemaphoreType.DMA,
        ],
        compiler_params=pltpu.CompilerParams(
            vmem_limit_bytes=60 * 1024 * 1024,
        ),
    )
    labels_flat, h_fin, c_fin = decode(
        encoded_lens.reshape(B, 1).astype(jnp.int32),
        encp,
        b.reshape(1, 4 * H),
        b_joint.reshape(1, J),
        bo_p,
        Wh.astype(jnp.bfloat16),
        W_pred.astype(jnp.bfloat16),
        wo_b,
        embwx,
    )
    labels = labels_flat.reshape(T, _U, B).transpose(2, 0, 1)
    return labels, h_fin, c_fin
